# baseline (speedup 1.0000x reference)
import functools

import jax
import jax.numpy as jnp
from jax import lax
from jax.experimental import pallas as pl
from jax.experimental.pallas import tpu as pltpu
from jax.experimental.pallas import tpu_sc as plsc

_B = 4096
_V = 100000
_HV0 = 50176
_HV1 = _V - _HV0
_NUM_CORES = 2
_NUM_SUBCORES = 16
_NW = _NUM_CORES * _NUM_SUBCORES
_D_BETAS = 10
_D_ORIENT = 3
_D_TRANSL = 3
_D_POSE = 69
_U_ORIENT = _D_POSE
_U_TRANSL = _U_ORIENT + _D_ORIENT
_U_BETAS = _U_TRANSL + _D_TRANSL
_N_UNITS = _U_BETAS + _D_BETAS


def _lookup_body(ids_hbm, betas_hbm, orient_hbm, transl_hbm, pose_hbm,
                 betas_out, orient_out, transl_out, pose_out,
                 idx_v, lo_idx, lo_pos, hi_idx, hi_pos,
                 row_a, row_b, out_v, bet_v, sem_a, sem_b):
    wid = lax.axis_index("s") * _NUM_CORES + lax.axis_index("c")
    iota = lax.iota(jnp.int32, 16)

    u0 = wid
    u1 = wid + _NW
    u2 = wid + 2 * _NW

    pltpu.sync_copy(ids_hbm, idx_v)

    cp_a = pltpu.async_copy(pose_hbm.at[u0].at[pl.ds(0, _HV0)], row_a, sem_a)
    cp_b = pltpu.async_copy(pose_hbm.at[u0].at[pl.ds(_HV0, _HV1)], row_b, sem_b)

    def part(k, c):
        nlo, nhi = c
        v = idx_v[pl.ds(k * 16, 16)]
        pos = iota + k * 16
        mlo = v < _HV0
        plsc.store_compressed(lo_idx.at[pl.ds(nlo, 16)], v, mask=mlo)
        plsc.store_compressed(lo_pos.at[pl.ds(nlo, 16)], pos, mask=mlo)
        mhi = jnp.logical_not(mlo)
        plsc.store_compressed(hi_idx.at[pl.ds(nhi, 16)], v - _HV0, mask=mhi)
        plsc.store_compressed(hi_pos.at[pl.ds(nhi, 16)], pos, mask=mhi)
        cnt = jnp.sum(mlo.astype(jnp.int32), axis=0)
        return nlo + cnt, nhi + (16 - cnt)

    n_lo, n_hi = lax.fori_loop(0, _B // 16, part, (0, 0))

    pltpu.sync_copy(betas_hbm.at[0], bet_v)

    def gather_half(row_buf, list_idx, list_pos, n):
        nspl = jnp.full((16,), n, jnp.int32)

        def gath(k, c):
            base = k * 16
            valid = (iota + base) < nspl
            lidx = list_idx[pl.ds(base, 16)]
            vals = plsc.load_gather(row_buf, [lidx], mask=valid)
            pos = list_pos[pl.ds(base, 16)]
            plsc.store_scatter(out_v, [pos], vals, mask=valid)
            return c

        lax.fori_loop(0, (n + 15) // 16, gath, 0)

    def wait_row(buf, sem, nwords):
        off = 0 if nwords == _HV0 else _HV0
        pltpu.make_async_copy(
            pose_hbm.at[0].at[pl.ds(off, nwords)], buf, sem).wait()

    def fire_lo(u):
        @pl.when(u < _U_ORIENT)
        def _():
            pltpu.async_copy(pose_hbm.at[u].at[pl.ds(0, _HV0)], row_a, sem_a)

        @pl.when((u >= _U_ORIENT) & (u < _U_TRANSL))
        def _():
            pltpu.async_copy(
                orient_hbm.at[u - _U_ORIENT].at[pl.ds(0, _HV0)], row_a, sem_a)

        @pl.when((u >= _U_TRANSL) & (u < _U_BETAS))
        def _():
            pltpu.async_copy(
                transl_hbm.at[u - _U_TRANSL].at[pl.ds(0, _HV0)], row_a, sem_a)

    def fire_hi(u):
        @pl.when(u < _U_ORIENT)
        def _():
            pltpu.async_copy(
                pose_hbm.at[u].at[pl.ds(_HV0, _HV1)], row_b, sem_b)

        @pl.when((u >= _U_ORIENT) & (u < _U_TRANSL))
        def _():
            pltpu.async_copy(
                orient_hbm.at[u - _U_ORIENT].at[pl.ds(_HV0, _HV1)], row_b,
                sem_b)

        @pl.when((u >= _U_TRANSL) & (u < _U_BETAS))
        def _():
            pltpu.async_copy(
                transl_hbm.at[u - _U_TRANSL].at[pl.ds(_HV0, _HV1)], row_b,
                sem_b)

    cp_a.wait()
    gather_half(row_a, lo_idx, lo_pos, n_lo)
    pltpu.async_copy(pose_hbm.at[u1].at[pl.ds(0, _HV0)], row_a, sem_a)
    cp_b.wait()
    gather_half(row_b, hi_idx, hi_pos, n_hi)
    pltpu.async_copy(pose_hbm.at[u1].at[pl.ds(_HV0, _HV1)], row_b, sem_b)
    pltpu.sync_copy(out_v, pose_out.at[u0])

    wait_row(row_a, sem_a, _HV0)
    gather_half(row_a, lo_idx, lo_pos, n_lo)
    fire_lo(u2)
    wait_row(row_b, sem_b, _HV1)
    gather_half(row_b, hi_idx, hi_pos, n_hi)
    fire_hi(u2)
    pltpu.sync_copy(out_v, pose_out.at[u1])

    def finish_unit(dst_row):
        wait_row(row_a, sem_a, _HV0)
        gather_half(row_a, lo_idx, lo_pos, n_lo)
        wait_row(row_b, sem_b, _HV1)
        gather_half(row_b, hi_idx, hi_pos, n_hi)
        pltpu.sync_copy(out_v, dst_row)

    @pl.when(u2 < _U_ORIENT)
    def _():
        finish_unit(pose_out.at[u2])

    @pl.when((u2 >= _U_ORIENT) & (u2 < _U_TRANSL))
    def _():
        finish_unit(orient_out.at[u2 - _U_ORIENT])

    @pl.when((u2 >= _U_TRANSL) & (u2 < _U_BETAS))
    def _():
        finish_unit(transl_out.at[u2 - _U_TRANSL])

    @pl.when((u2 >= _U_BETAS) & (u2 < _N_UNITS))
    def _():
        j = u2 - _U_BETAS
        vals = plsc.load_gather(bet_v, [jnp.full((16,), j, jnp.int32)])

        def splat(k, c):
            for s in range(8):
                out_v[pl.ds(k * 128 + s * 16, 16)] = vals
            return c

        lax.fori_loop(0, _B // 128, splat, 0)
        pltpu.sync_copy(out_v, betas_out.at[j])


@jax.jit
def kernel(frame_ids, betas_w, global_orient_w, transl_w, body_pose_w):
    mesh = plsc.VectorSubcoreMesh(
        core_axis_name="c", subcore_axis_name="s",
        num_cores=_NUM_CORES, num_subcores=_NUM_SUBCORES)
    run = pl.kernel(
        _lookup_body,
        out_type=(
            jax.ShapeDtypeStruct((_D_BETAS, _B), jnp.float32),
            jax.ShapeDtypeStruct((_D_ORIENT, _B), jnp.float32),
            jax.ShapeDtypeStruct((_D_TRANSL, _B), jnp.float32),
            jax.ShapeDtypeStruct((_D_POSE, _B), jnp.float32),
        ),
        mesh=mesh,
        compiler_params=pltpu.CompilerParams(
            use_tc_tiling_on_sc=True, needs_layout_passes=False),
        scratch_types=[
            pltpu.VMEM((_B,), jnp.int32),
            pltpu.VMEM((_B + 16,), jnp.int32),
            pltpu.VMEM((_B + 16,), jnp.int32),
            pltpu.VMEM((_B + 16,), jnp.int32),
            pltpu.VMEM((_B + 16,), jnp.int32),
            pltpu.VMEM((_HV0,), jnp.float32),
            pltpu.VMEM((_HV1,), jnp.float32),
            pltpu.VMEM((_B,), jnp.float32),
            pltpu.VMEM((10,), jnp.float32),
            pltpu.SemaphoreType.DMA,
            pltpu.SemaphoreType.DMA,
        ],
    )
    betas_t, orient_t, transl_t, pose_t = run(
        frame_ids.astype(jnp.int32), betas_w,
        jnp.swapaxes(global_orient_w, 0, 1),
        jnp.swapaxes(transl_w, 0, 1),
        jnp.swapaxes(body_pose_w, 0, 1))
    return (jnp.swapaxes(betas_t, 0, 1), jnp.swapaxes(orient_t, 0, 1),
            jnp.swapaxes(transl_t, 0, 1), jnp.swapaxes(pose_t, 0, 1))


# speedup vs baseline: 1.0075x; 1.0075x over previous
import functools

import jax
import jax.numpy as jnp
from jax import lax
from jax.experimental import pallas as pl
from jax.experimental.pallas import tpu as pltpu
from jax.experimental.pallas import tpu_sc as plsc

_B = 4096
_V = 100000
_HV0 = 50176
_HV1 = _V - _HV0
_NUM_CORES = 2
_NUM_SUBCORES = 16
_NW = _NUM_CORES * _NUM_SUBCORES
_D_BETAS = 10
_D_ORIENT = 3
_D_TRANSL = 3
_D_POSE = 69
_U_ORIENT = _D_POSE
_U_TRANSL = _U_ORIENT + _D_ORIENT
_U_BETAS = _U_TRANSL + _D_TRANSL
_N_UNITS = _U_BETAS + _D_BETAS


def _lookup_body(ids_hbm, betas_hbm, orient_hbm, transl_hbm, pose_hbm,
                 betas_out, orient_out, transl_out, pose_out,
                 idx_v, lo_idx, lo_pos, hi_idx, hi_pos,
                 row_a, row_b, out_v, bet_v, sem_a, sem_b):
    cid = lax.axis_index("c")
    sid = lax.axis_index("s")
    iota = lax.iota(jnp.int32, 16)

    u0 = cid * _NUM_SUBCORES + sid
    u1 = u0 + _NW
    u2 = 2 * _NW + sid * _NUM_CORES + cid

    pltpu.sync_copy(ids_hbm, idx_v)

    cp_a = pltpu.async_copy(pose_hbm.at[u0].at[pl.ds(0, _HV0)], row_a, sem_a)
    cp_b = pltpu.async_copy(pose_hbm.at[u0].at[pl.ds(_HV0, _HV1)], row_b, sem_b)

    def part(k, c):
        nlo, nhi = c
        v = idx_v[pl.ds(k * 16, 16)]
        pos = iota + k * 16
        mlo = v < _HV0
        plsc.store_compressed(lo_idx.at[pl.ds(nlo, 16)], v, mask=mlo)
        plsc.store_compressed(lo_pos.at[pl.ds(nlo, 16)], pos, mask=mlo)
        mhi = jnp.logical_not(mlo)
        plsc.store_compressed(hi_idx.at[pl.ds(nhi, 16)], v - _HV0, mask=mhi)
        plsc.store_compressed(hi_pos.at[pl.ds(nhi, 16)], pos, mask=mhi)
        cnt = jnp.sum(mlo.astype(jnp.int32), axis=0)
        return nlo + cnt, nhi + (16 - cnt)

    n_lo, n_hi = lax.fori_loop(0, _B // 16, part, (0, 0))

    pltpu.sync_copy(betas_hbm.at[0], bet_v)

    def gather_half(row_buf, list_idx, list_pos, n):
        nspl = jnp.full((16,), n, jnp.int32)

        def gath(k, c):
            base = k * 16
            valid = (iota + base) < nspl
            lidx = list_idx[pl.ds(base, 16)]
            vals = plsc.load_gather(row_buf, [lidx], mask=valid)
            pos = list_pos[pl.ds(base, 16)]
            plsc.store_scatter(out_v, [pos], vals, mask=valid)
            return c

        lax.fori_loop(0, (n + 15) // 16, gath, 0)

    def wait_row(buf, sem, nwords):
        off = 0 if nwords == _HV0 else _HV0
        pltpu.make_async_copy(
            pose_hbm.at[0].at[pl.ds(off, nwords)], buf, sem).wait()

    def fire_lo(u):
        @pl.when(u < _U_ORIENT)
        def _():
            pltpu.async_copy(pose_hbm.at[u].at[pl.ds(0, _HV0)], row_a, sem_a)

        @pl.when((u >= _U_ORIENT) & (u < _U_TRANSL))
        def _():
            pltpu.async_copy(
                orient_hbm.at[u - _U_ORIENT].at[pl.ds(0, _HV0)], row_a, sem_a)

        @pl.when((u >= _U_TRANSL) & (u < _U_BETAS))
        def _():
            pltpu.async_copy(
                transl_hbm.at[u - _U_TRANSL].at[pl.ds(0, _HV0)], row_a, sem_a)

    def fire_hi(u):
        @pl.when(u < _U_ORIENT)
        def _():
            pltpu.async_copy(
                pose_hbm.at[u].at[pl.ds(_HV0, _HV1)], row_b, sem_b)

        @pl.when((u >= _U_ORIENT) & (u < _U_TRANSL))
        def _():
            pltpu.async_copy(
                orient_hbm.at[u - _U_ORIENT].at[pl.ds(_HV0, _HV1)], row_b,
                sem_b)

        @pl.when((u >= _U_TRANSL) & (u < _U_BETAS))
        def _():
            pltpu.async_copy(
                transl_hbm.at[u - _U_TRANSL].at[pl.ds(_HV0, _HV1)], row_b,
                sem_b)

    cp_a.wait()
    gather_half(row_a, lo_idx, lo_pos, n_lo)
    pltpu.async_copy(pose_hbm.at[u1].at[pl.ds(0, _HV0)], row_a, sem_a)
    cp_b.wait()
    gather_half(row_b, hi_idx, hi_pos, n_hi)
    pltpu.async_copy(pose_hbm.at[u1].at[pl.ds(_HV0, _HV1)], row_b, sem_b)
    pltpu.sync_copy(out_v, pose_out.at[u0])

    wait_row(row_a, sem_a, _HV0)
    gather_half(row_a, lo_idx, lo_pos, n_lo)
    fire_lo(u2)
    wait_row(row_b, sem_b, _HV1)
    gather_half(row_b, hi_idx, hi_pos, n_hi)
    fire_hi(u2)
    pltpu.sync_copy(out_v, pose_out.at[u1])

    def finish_unit(dst_row):
        wait_row(row_a, sem_a, _HV0)
        gather_half(row_a, lo_idx, lo_pos, n_lo)
        wait_row(row_b, sem_b, _HV1)
        gather_half(row_b, hi_idx, hi_pos, n_hi)
        pltpu.sync_copy(out_v, dst_row)

    @pl.when(u2 < _U_ORIENT)
    def _():
        finish_unit(pose_out.at[u2])

    @pl.when((u2 >= _U_ORIENT) & (u2 < _U_TRANSL))
    def _():
        finish_unit(orient_out.at[u2 - _U_ORIENT])

    @pl.when((u2 >= _U_TRANSL) & (u2 < _U_BETAS))
    def _():
        finish_unit(transl_out.at[u2 - _U_TRANSL])

    @pl.when((u2 >= _U_BETAS) & (u2 < _N_UNITS))
    def _():
        j = u2 - _U_BETAS
        vals = plsc.load_gather(bet_v, [jnp.full((16,), j, jnp.int32)])

        def splat(k, c):
            for s in range(8):
                out_v[pl.ds(k * 128 + s * 16, 16)] = vals
            return c

        lax.fori_loop(0, _B // 128, splat, 0)
        pltpu.sync_copy(out_v, betas_out.at[j])


@jax.jit
def kernel(frame_ids, betas_w, global_orient_w, transl_w, body_pose_w):
    mesh = plsc.VectorSubcoreMesh(
        core_axis_name="c", subcore_axis_name="s",
        num_cores=_NUM_CORES, num_subcores=_NUM_SUBCORES)
    run = pl.kernel(
        _lookup_body,
        out_type=(
            jax.ShapeDtypeStruct((_D_BETAS, _B), jnp.float32),
            jax.ShapeDtypeStruct((_D_ORIENT, _B), jnp.float32),
            jax.ShapeDtypeStruct((_D_TRANSL, _B), jnp.float32),
            jax.ShapeDtypeStruct((_D_POSE, _B), jnp.float32),
        ),
        mesh=mesh,
        compiler_params=pltpu.CompilerParams(
            use_tc_tiling_on_sc=True, needs_layout_passes=False),
        scratch_types=[
            pltpu.VMEM((_B,), jnp.int32),
            pltpu.VMEM((_B + 16,), jnp.int32),
            pltpu.VMEM((_B + 16,), jnp.int32),
            pltpu.VMEM((_B + 16,), jnp.int32),
            pltpu.VMEM((_B + 16,), jnp.int32),
            pltpu.VMEM((_HV0,), jnp.float32),
            pltpu.VMEM((_HV1,), jnp.float32),
            pltpu.VMEM((_B,), jnp.float32),
            pltpu.VMEM((10,), jnp.float32),
            pltpu.SemaphoreType.DMA,
            pltpu.SemaphoreType.DMA,
        ],
    )
    betas_t, orient_t, transl_t, pose_t = run(
        frame_ids.astype(jnp.int32), betas_w,
        jnp.swapaxes(global_orient_w, 0, 1),
        jnp.swapaxes(transl_w, 0, 1),
        jnp.swapaxes(body_pose_w, 0, 1))
    return (jnp.swapaxes(betas_t, 0, 1), jnp.swapaxes(orient_t, 0, 1),
            jnp.swapaxes(transl_t, 0, 1), jnp.swapaxes(pose_t, 0, 1))
